# Initial kernel scaffold; baseline (speedup 1.0000x reference)
import jax
import jax.numpy as jnp
from jax.experimental import pallas as pl


def kernel(inputs, edge_index, W1, b1, W2, b2, Wf, bf):
    raise NotImplementedError("write your pallas kernel here")


# speedup vs baseline: 25.0250x; 25.0250x over previous
import jax
import jax.numpy as jnp
from jax import lax
from jax.experimental import pallas as pl
from jax.experimental.pallas import tpu as pltpu
from jax.experimental.pallas import tpu_sc as plsc

NC = 2
NS = 16
NW = NC * NS
LANES = 16
C = 128

_MESH = dict(core_axis_name="c", subcore_axis_name="s", num_cores=NC,
             num_subcores=NS)


def _deg_partials(dst2d, n):
  nrows = dst2d.shape[0]
  cw = dst2d.shape[2]
  cpt = nrows // NW
  extra = nrows - cpt * NW

  def body(dst_hbm, parts_hbm, hist, dslab, dx):
    cid = lax.axis_index("c")
    sid = lax.axis_index("s")
    wid = cid * NS + sid
    zero16 = jnp.zeros((LANES,), jnp.float32)
    ones16 = jnp.ones((LANES,), jnp.float32)

    pltpu.sync_copy(dst_hbm.at[pl.ds(wid * cpt, cpt)], dslab)

    def zloop(i, _):
      hist[pl.ds(i * LANES, LANES)] = zero16
      return 0
    lax.fori_loop(0, n // LANES, zloop, 0)

    def chunk(k, _):
      for j in range(cw // LANES):
        iv = dslab[k, 0, pl.ds(j * LANES, LANES)]
        plsc.addupdate_scatter(hist, [iv], ones16)
      return 0
    lax.fori_loop(0, cpt, chunk, 0)

    if extra:
      @pl.when(wid < extra)
      def _():
        pltpu.sync_copy(dst_hbm.at[pl.ds(NW * cpt + wid, 1)], dx)
        for j in range(cw // LANES):
          iv = dx[0, 0, pl.ds(j * LANES, LANES)]
          plsc.addupdate_scatter(hist, [iv], ones16)

    pltpu.sync_copy(hist, parts_hbm.at[wid, 0])

  return pl.kernel(
      body,
      out_type=jax.ShapeDtypeStruct((NW, 1, n), jnp.float32),
      mesh=plsc.VectorSubcoreMesh(**_MESH),
      compiler_params=pltpu.CompilerParams(needs_layout_passes=False),
      scratch_types=[
          pltpu.VMEM((n,), jnp.float32),
          pltpu.VMEM((cpt, 1, cw), jnp.int32),
          pltpu.VMEM((1, 1, cw), jnp.int32),
      ],
  )(dst2d)


def _spmm_raw(x, dst2d, src2d):
  n, h = x.shape
  nrows = dst2d.shape[0]
  cw = dst2d.shape[2]
  cpt = nrows // NW
  extra = nrows - cpt * NW
  rpt = (n // NS) // 8 * 8
  nrem = n - NS * rpt
  assert cpt % 2 == 0 and extra <= NW

  def body(x_hbm, dst_hbm, src_hbm, out_hbm, acc, sslab, didx0, didx1,
           rows0, rows1, sx, dx, slabsem, g0, g1, d0, d1):
    cid = lax.axis_index("c")
    sid = lax.axis_index("s")
    wid = cid * NS + sid
    rb = sid * rpt
    zero16 = jnp.zeros((LANES,), jnp.float32)
    rows = (rows0, rows1)
    gsem = (g0, g1)
    didx = (didx0, didx1)
    dsem = (d0, d1)

    def wait_gather(k, b):
      pltpu.make_async_copy(x_hbm.at[sslab.at[k, 0]], rows[b], gsem[b]).wait()

    def issue_gather(k, b):
      pltpu.async_copy(x_hbm.at[sslab.at[k, 0]], rows[b], gsem[b])

    def issue_didx(k, b):
      pltpu.async_copy(dst_hbm.at[pl.ds(wid * cpt + k, 1)], didx[b], dsem[b])

    def wait_didx(k, b):
      pltpu.make_async_copy(dst_hbm.at[pl.ds(wid * cpt + k, 1)], didx[b],
                            dsem[b]).wait()

    cs = pltpu.async_copy(src_hbm.at[pl.ds(wid * cpt, cpt)], sslab, slabsem)
    pltpu.sync_copy(dst_hbm.at[pl.ds(wid * cpt, 1)], didx0)

    vpr = h // LANES

    def zloop(i, _):
      rows0[i // vpr, pl.ds((i % vpr) * LANES, LANES)] = zero16
      return 0
    lax.fori_loop(0, cw * vpr, zloop, 0)

    nz = rpt // cw
    for j in range(nz):
      pltpu.sync_copy(rows0, acc.at[pl.ds(rb + j * cw, cw)])
    rem = rpt - nz * cw
    if rem:
      pltpu.sync_copy(rows0.at[pl.ds(0, rem)], acc.at[pl.ds(rb + nz * cw, rem)])
    if nrem:
      @pl.when(sid == NS - 1)
      def _():
        pltpu.sync_copy(rows0.at[pl.ds(0, nrem)],
                        acc.at[pl.ds(NS * rpt, nrem)])
    cs.wait()
    plsc.subcore_barrier()

    issue_gather(0, 0)

    def pair(i, _):
      for b in range(2):
        k = 2 * i + b
        wait_gather(k, b)

        @pl.when(k + 1 < cpt)
        def _():
          issue_gather(k + 1, 1 - b)
          issue_didx(k + 1, 1 - b)
        if b == 0:
          @pl.when(k > 0)
          def _():
            wait_didx(k, b)
        else:
          wait_didx(k, b)
        pltpu.sync_copy(rows[b], acc.at[didx[b].at[0, 0]], add=True)
      return 0
    lax.fori_loop(0, cpt // 2, pair, 0)

    if extra:
      @pl.when(wid < extra)
      def _():
        pltpu.sync_copy(src_hbm.at[pl.ds(NW * cpt + wid, 1)], sx)
        pltpu.sync_copy(dst_hbm.at[pl.ds(NW * cpt + wid, 1)], dx)
        pltpu.async_copy(x_hbm.at[sx.at[0, 0]], rows0, g0).wait()
        pltpu.sync_copy(rows0, acc.at[dx.at[0, 0]], add=True)

    plsc.subcore_barrier()
    pltpu.sync_copy(acc.at[pl.ds(rb, rpt)], out_hbm.at[cid, pl.ds(rb, rpt)])
    if nrem:
      @pl.when(sid == NS - 1)
      def _():
        pltpu.sync_copy(acc.at[pl.ds(NS * rpt, nrem)],
                        out_hbm.at[cid, pl.ds(NS * rpt, nrem)])

  return pl.kernel(
      body,
      out_type=jax.ShapeDtypeStruct((NC, n, h), jnp.float32),
      mesh=plsc.VectorSubcoreMesh(**_MESH),
      compiler_params=pltpu.CompilerParams(needs_layout_passes=False,
                                           use_tc_tiling_on_sc=False),
      scratch_types=[
          pltpu.VMEM_SHARED((n, h), jnp.float32),
          pltpu.VMEM((cpt, 1, cw), jnp.int32),
          pltpu.VMEM((1, 1, cw), jnp.int32),
          pltpu.VMEM((1, 1, cw), jnp.int32),
          pltpu.VMEM((cw, h), jnp.float32),
          pltpu.VMEM((cw, h), jnp.float32),
          pltpu.VMEM((1, 1, cw), jnp.int32),
          pltpu.VMEM((1, 1, cw), jnp.int32),
      ] + [pltpu.SemaphoreType.DMA] * 5,
  )(x, dst2d, src2d)


def _prep(parts, x, bn=2048):
  n, h = x.shape

  def body(p_ref, x_ref, d_ref, o_ref):
    deg = jnp.sum(p_ref[...], axis=0, keepdims=True)
    dinv = jnp.where(deg > 0, lax.rsqrt(deg), 0.0)
    dcol = jnp.transpose(dinv, (1, 0))
    d_ref[...] = dcol
    o_ref[...] = dcol * x_ref[...]

  return pl.pallas_call(
      body,
      grid=(pl.cdiv(n, bn),),
      in_specs=[
          pl.BlockSpec((NW, bn), lambda i: (0, i)),
          pl.BlockSpec((bn, h), lambda i: (i, 0)),
      ],
      out_specs=[
          pl.BlockSpec((bn, 1), lambda i: (i, 0)),
          pl.BlockSpec((bn, h), lambda i: (i, 0)),
      ],
      out_shape=[
          jax.ShapeDtypeStruct((n, 1), jnp.float32),
          jax.ShapeDtypeStruct((n, h), jnp.float32),
      ],
  )(parts, x)


def _mid_layer(raw, dinv_col, b1_row, w2, bn=2000):
  n, h = raw.shape[1], raw.shape[2]

  def body(r_ref, d_ref, b_ref, w_ref, o_ref):
    d = d_ref[...]
    hidden = jnp.maximum(d * (r_ref[0] + r_ref[1]) + b_ref[...], 0.0)
    o_ref[...] = d * jnp.dot(hidden, w_ref[...],
                             preferred_element_type=jnp.float32)

  return pl.pallas_call(
      body,
      grid=(n // bn,),
      in_specs=[
          pl.BlockSpec((NC, bn, h), lambda i: (0, i, 0)),
          pl.BlockSpec((bn, 1), lambda i: (i, 0)),
          pl.BlockSpec((1, h), lambda i: (0, 0)),
          pl.BlockSpec((h, h), lambda i: (0, 0)),
      ],
      out_specs=pl.BlockSpec((bn, h), lambda i: (i, 0)),
      out_shape=jax.ShapeDtypeStruct((n, h), jnp.float32),
  )(raw, dinv_col, b1_row, w2)


def _head(raw, dinv_col, b2_row, wf, bf_2d, bn=2000):
  n, h = raw.shape[1], raw.shape[2]

  def body(r_ref, d_ref, b_ref, w_ref, bf_ref, o_ref):
    d = d_ref[...]
    hidden = jnp.maximum(d * (r_ref[0] + r_ref[1]) + b_ref[...], 0.0)
    o_ref[...] = jnp.dot(hidden, w_ref[...],
                         preferred_element_type=jnp.float32) + bf_ref[...]

  return pl.pallas_call(
      body,
      grid=(n // bn,),
      in_specs=[
          pl.BlockSpec((NC, bn, h), lambda i: (0, i, 0)),
          pl.BlockSpec((bn, 1), lambda i: (i, 0)),
          pl.BlockSpec((1, h), lambda i: (0, 0)),
          pl.BlockSpec((h, 1), lambda i: (0, 0)),
          pl.BlockSpec((1, 1), lambda i: (0, 0)),
      ],
      out_specs=pl.BlockSpec((bn, 1), lambda i: (i, 0)),
      out_shape=jax.ShapeDtypeStruct((n, 1), jnp.float32),
  )(raw, dinv_col, b2_row, wf, bf_2d)


def kernel(inputs, edge_index, W1, b1, W2, b2, Wf, bf):
  del inputs
  n, h = W1.shape
  e = edge_index.shape[1]
  assert e % C == 0
  dst2d = edge_index[0].reshape(e // C, 1, C)
  src2d = edge_index[1].reshape(e // C, 1, C)

  parts = _deg_partials(dst2d, n)
  dinv_col, x1 = _prep(parts.reshape(NW, n), W1)
  raw1 = _spmm_raw(x1, dst2d, src2d)
  x2 = _mid_layer(raw1, dinv_col, b1.reshape(1, h), W2)
  raw2 = _spmm_raw(x2, dst2d, src2d)
  out = _head(raw2, dinv_col, b2.reshape(1, h), Wf, bf.reshape(1, 1))
  return out.reshape(1, n)
